# Initial kernel scaffold; baseline (speedup 1.0000x reference)
import jax
import jax.numpy as jnp
from jax.experimental import pallas as pl


def kernel(x, edge_index, W1, b1, W2, b2):
    raise NotImplementedError("write your pallas kernel here")


# speedup vs baseline: 2.3498x; 2.3498x over previous
import functools

import jax
import jax.numpy as jnp
from jax import lax
from jax.experimental import pallas as pl
from jax.experimental.pallas import tpu as pltpu
from jax.experimental.pallas import tpu_sc as plsc

N = 10000
D = 128
E = 320000

NC = 2
NS = 16
NW = NC * NS
CH = 128
EPW = 10240
E_PAD = EPW * NW
NCH = EPW // CH
N_PAD = 10240
RPT = N_PAD // NS

BLK = 1000
GRID = N // BLK

_mesh = plsc.VectorSubcoreMesh(core_axis_name="c", subcore_axis_name="s")


@functools.partial(
    pl.kernel,
    mesh=_mesh,
    out_type=(
        jax.ShapeDtypeStruct((NC * N_PAD,), jnp.float32),
        jax.ShapeDtypeStruct((NC * N_PAD,), jnp.float32),
    ),
    scratch_types=[
        pltpu.VMEM((CH,), jnp.int32),
        pltpu.VMEM((CH,), jnp.int32),
        pltpu.VMEM((CH,), jnp.float32),
        pltpu.VMEM((CH,), jnp.float32),
        pltpu.VMEM_SHARED((N_PAD,), jnp.float32),
        pltpu.VMEM_SHARED((N_PAD,), jnp.float32),
    ],
)
def _deg_kernel(src_h, dst_h, deg_o_h, deg_i_h,
                sidx, didx, ones_v, zeros_v, deg_o_s, deg_i_s):
    c = lax.axis_index("c")
    s = lax.axis_index("s")
    wid = s * NC + c
    for i in range(CH // 16):
        ones_v[pl.ds(i * 16, 16)] = jnp.ones((16,), jnp.float32)
        zeros_v[pl.ds(i * 16, 16)] = jnp.zeros((16,), jnp.float32)
    for j in range(RPT // CH):
        pltpu.sync_copy(zeros_v, deg_o_s.at[pl.ds(s * RPT + j * CH, CH)])
        pltpu.sync_copy(zeros_v, deg_i_s.at[pl.ds(s * RPT + j * CH, CH)])
    plsc.subcore_barrier()

    base0 = wid * EPW

    def body(i, carry):
        base = base0 + i * CH
        pltpu.sync_copy(src_h.at[pl.ds(base, CH)], sidx)
        pltpu.sync_copy(dst_h.at[pl.ds(base, CH)], didx)
        pltpu.sync_copy(ones_v, deg_o_s.at[sidx], add=True)
        pltpu.sync_copy(ones_v, deg_i_s.at[didx], add=True)
        return carry

    lax.fori_loop(0, NCH, body, 0)
    plsc.subcore_barrier()
    pltpu.sync_copy(deg_o_s.at[pl.ds(s * RPT, RPT)],
                    deg_o_h.at[pl.ds(c * N_PAD + s * RPT, RPT)])
    pltpu.sync_copy(deg_i_s.at[pl.ds(s * RPT, RPT)],
                    deg_i_h.at[pl.ds(c * N_PAD + s * RPT, RPT)])


@functools.partial(
    pl.kernel,
    mesh=_mesh,
    out_type=jax.ShapeDtypeStruct((NC, N_PAD, D), jnp.float32),
    scratch_types=[
        pltpu.VMEM((CH,), jnp.int32),
        pltpu.VMEM((CH,), jnp.int32),
        pltpu.VMEM((CH, D), jnp.float32),
        pltpu.VMEM_SHARED((N_PAD, D), jnp.float32),
        pltpu.SemaphoreType.DMA,
    ],
)
def _agg_kernel(h_h, src_h, dst_h, zeros_h, out_h, sidx, didx, rows, accum, sem):
    c = lax.axis_index("c")
    s = lax.axis_index("s")
    wid = s * NC + c
    pltpu.sync_copy(zeros_h.at[pl.ds(s * RPT, RPT)], accum.at[pl.ds(s * RPT, RPT)])
    plsc.subcore_barrier()

    base0 = wid * EPW

    def body(i, carry):
        base = base0 + i * CH
        pltpu.sync_copy(src_h.at[pl.ds(base, CH)], sidx)
        pltpu.async_copy(h_h.at[sidx], rows, sem).wait()
        pltpu.sync_copy(dst_h.at[pl.ds(base, CH)], didx)
        pltpu.sync_copy(rows, accum.at[didx], add=True)
        return carry

    lax.fori_loop(0, NCH, body, 0)
    plsc.subcore_barrier()
    pltpu.sync_copy(accum.at[pl.ds(s * RPT, RPT)], out_h.at[c, pl.ds(s * RPT, RPT)])


def _tc1_body(x_ref, do0, do1, di0, di1, hs_ref, ns_ref, nd_ref):
    deg_o = do0[0] + do1[0]
    deg_i = di0[0] + di1[0]
    ns = lax.rsqrt(jnp.maximum(deg_o, 1.0))
    nd = lax.rsqrt(jnp.maximum(deg_i, 1.0))
    ns_ref[...] = ns
    nd_ref[...] = nd
    hs_ref[...] = x_ref[...] * ns


def _tc1(x, deg_o_p, deg_i_p):
    return pl.pallas_call(
        _tc1_body,
        grid=(GRID,),
        in_specs=[
            pl.BlockSpec((BLK, D), lambda i: (i, 0)),
            pl.BlockSpec((1, BLK, 1), lambda i: (0, i, 0)),
            pl.BlockSpec((1, BLK, 1), lambda i: (1, i, 0)),
            pl.BlockSpec((1, BLK, 1), lambda i: (0, i, 0)),
            pl.BlockSpec((1, BLK, 1), lambda i: (1, i, 0)),
        ],
        out_specs=[
            pl.BlockSpec((BLK, D), lambda i: (i, 0)),
            pl.BlockSpec((BLK, 1), lambda i: (i, 0)),
            pl.BlockSpec((BLK, 1), lambda i: (i, 0)),
        ],
        out_shape=[
            jax.ShapeDtypeStruct((N, D), jnp.float32),
            jax.ShapeDtypeStruct((N, 1), jnp.float32),
            jax.ShapeDtypeStruct((N, 1), jnp.float32),
        ],
    )(x, deg_o_p, deg_o_p, deg_i_p, deg_i_p)


def _tc2_body(p_ref0, p_ref1, nd_ref, ns_ref, w_ref, b_ref, out_ref):
    m = (p_ref0[0] + p_ref1[0]) * nd_ref[...]
    h = jnp.dot(m, w_ref[...], preferred_element_type=jnp.float32) + b_ref[...]
    out_ref[...] = jnp.maximum(h, 0.0) * ns_ref[...]


def _tc2(m_p, nd, ns, w, b):
    return pl.pallas_call(
        _tc2_body,
        grid=(GRID,),
        in_specs=[
            pl.BlockSpec((1, BLK, D), lambda i: (0, i, 0)),
            pl.BlockSpec((1, BLK, D), lambda i: (1, i, 0)),
            pl.BlockSpec((BLK, 1), lambda i: (i, 0)),
            pl.BlockSpec((BLK, 1), lambda i: (i, 0)),
            pl.BlockSpec((D, D), lambda i: (0, 0)),
            pl.BlockSpec((1, D), lambda i: (0, 0)),
        ],
        out_specs=pl.BlockSpec((BLK, D), lambda i: (i, 0)),
        out_shape=jax.ShapeDtypeStruct((N, D), jnp.float32),
    )(m_p, m_p, nd, ns, w, b)


def _tc3_body(p_ref0, p_ref1, nd_ref, w_ref, b_ref, out_ref, acc_vec, acc_nrm):
    i = pl.program_id(0)
    m = (p_ref0[0] + p_ref1[0]) * nd_ref[...]
    h = jnp.dot(m, w_ref[...], preferred_element_type=jnp.float32) + b_ref[...]
    blk_sum = jnp.sum(h, axis=0, keepdims=True)
    blk_nrm = jnp.sum(jnp.sqrt(jnp.sum(h * h, axis=1)))

    @pl.when(i == 0)
    def _():
        acc_vec[...] = jnp.zeros_like(acc_vec)
        acc_nrm[0] = 0.0

    acc_vec[...] += blk_sum
    acc_nrm[0] += blk_nrm

    @pl.when(i == pl.num_programs(0) - 1)
    def _():
        factor = jnp.sqrt(jnp.float32(D)) * jnp.float32(N) / acc_nrm[0]
        out_ref[...] = acc_vec[...] * factor


def _tc3(m_p, nd, w, b):
    return pl.pallas_call(
        _tc3_body,
        grid=(GRID,),
        in_specs=[
            pl.BlockSpec((1, BLK, D), lambda i: (0, i, 0)),
            pl.BlockSpec((1, BLK, D), lambda i: (1, i, 0)),
            pl.BlockSpec((BLK, 1), lambda i: (i, 0)),
            pl.BlockSpec((D, D), lambda i: (0, 0)),
            pl.BlockSpec((1, D), lambda i: (0, 0)),
        ],
        out_specs=pl.BlockSpec((1, D), lambda i: (0, 0)),
        out_shape=jax.ShapeDtypeStruct((1, D), jnp.float32),
        scratch_shapes=[
            pltpu.VMEM((1, D), jnp.float32),
            pltpu.SMEM((1,), jnp.float32),
        ],
    )(m_p, m_p, nd, w, b)


def kernel(x, edge_index, W1, b1, W2, b2):
    ei = edge_index.astype(jnp.int32)
    src = ei[0]
    dst = ei[1]
    pad = E_PAD - E
    src_g = jnp.concatenate([src, jnp.zeros((pad,), jnp.int32)])
    src_d = jnp.concatenate([src, jnp.full((pad,), N, jnp.int32)])
    dst_p = jnp.concatenate([dst, jnp.full((pad,), N, jnp.int32)])

    zeros_d = jnp.zeros((N_PAD, D), jnp.float32)

    deg_o_f, deg_i_f = _deg_kernel(src_d, dst_p)
    deg_o_p = deg_o_f.reshape(NC, N_PAD, 1)
    deg_i_p = deg_i_f.reshape(NC, N_PAD, 1)
    h1s, ns, nd = _tc1(x, deg_o_p, deg_i_p)
    m1_p = _agg_kernel(h1s, src_g, dst_p, zeros_d)
    h2s = _tc2(m1_p, nd, ns, W1, b1.reshape(1, D))
    m2_p = _agg_kernel(h2s, src_g, dst_p, zeros_d)
    out = _tc3(m2_p, nd, W2, b2.reshape(1, D))
    return out


# speedup vs baseline: 2.8156x; 1.1982x over previous
import functools

import jax
import jax.numpy as jnp
from jax import lax
from jax.experimental import pallas as pl
from jax.experimental.pallas import tpu as pltpu
from jax.experimental.pallas import tpu_sc as plsc

N = 10000
D = 128
E = 320000

NC = 2
NS = 16
NW = NC * NS
CH = 128
EPW = 10240
E_PAD = EPW * NW
NCH = EPW // CH
N_PAD = 10240
RPT = N_PAD // NS

BLK = 1000
GRID = N // BLK

_mesh = plsc.VectorSubcoreMesh(core_axis_name="c", subcore_axis_name="s")


@functools.partial(
    pl.kernel,
    mesh=_mesh,
    out_type=(
        jax.ShapeDtypeStruct((NC * N_PAD,), jnp.float32),
        jax.ShapeDtypeStruct((NC * N_PAD,), jnp.float32),
    ),
    scratch_types=[
        pltpu.VMEM((CH,), jnp.int32),
        pltpu.VMEM((CH,), jnp.int32),
        pltpu.VMEM((CH,), jnp.float32),
        pltpu.VMEM((CH,), jnp.float32),
        pltpu.VMEM_SHARED((N_PAD,), jnp.float32),
        pltpu.VMEM_SHARED((N_PAD,), jnp.float32),
    ],
)
def _deg_kernel(src_h, dst_h, deg_o_h, deg_i_h,
                sidx, didx, ones_v, zeros_v, deg_o_s, deg_i_s):
    c = lax.axis_index("c")
    s = lax.axis_index("s")
    wid = s * NC + c
    for i in range(CH // 16):
        ones_v[pl.ds(i * 16, 16)] = jnp.ones((16,), jnp.float32)
        zeros_v[pl.ds(i * 16, 16)] = jnp.zeros((16,), jnp.float32)
    for j in range(RPT // CH):
        pltpu.sync_copy(zeros_v, deg_o_s.at[pl.ds(s * RPT + j * CH, CH)])
        pltpu.sync_copy(zeros_v, deg_i_s.at[pl.ds(s * RPT + j * CH, CH)])
    plsc.subcore_barrier()

    base0 = wid * EPW

    def body(i, carry):
        base = base0 + i * CH
        pltpu.sync_copy(src_h.at[pl.ds(base, CH)], sidx)
        pltpu.sync_copy(dst_h.at[pl.ds(base, CH)], didx)
        pltpu.sync_copy(ones_v, deg_o_s.at[sidx], add=True)
        pltpu.sync_copy(ones_v, deg_i_s.at[didx], add=True)
        return carry

    lax.fori_loop(0, NCH, body, 0)
    plsc.subcore_barrier()
    pltpu.sync_copy(deg_o_s.at[pl.ds(s * RPT, RPT)],
                    deg_o_h.at[pl.ds(c * N_PAD + s * RPT, RPT)])
    pltpu.sync_copy(deg_i_s.at[pl.ds(s * RPT, RPT)],
                    deg_i_h.at[pl.ds(c * N_PAD + s * RPT, RPT)])


@functools.partial(
    pl.kernel,
    mesh=_mesh,
    out_type=jax.ShapeDtypeStruct((NC, N_PAD, D), jnp.float32),
    scratch_types=[
        pltpu.VMEM((2, CH), jnp.int32),
        pltpu.VMEM((2, CH), jnp.int32),
        pltpu.VMEM((2, CH, D), jnp.float32),
        pltpu.VMEM_SHARED((N_PAD, D), jnp.float32),
        pltpu.SemaphoreType.DMA,
        pltpu.SemaphoreType.DMA,
    ],
)
def _agg_kernel(h_h, src_h, dst_h, zeros_h, out_h, sidx, didx, rows, accum,
                sem0, sem1):
    c = lax.axis_index("c")
    s = lax.axis_index("s")
    wid = s * NC + c
    pltpu.sync_copy(zeros_h.at[pl.ds(s * RPT, RPT)], accum.at[pl.ds(s * RPT, RPT)])
    plsc.subcore_barrier()

    base0 = wid * EPW
    sems = (sem0, sem1)

    for b in range(2):
        pltpu.sync_copy(src_h.at[pl.ds(base0 + b * CH, CH)], sidx.at[b])
        pltpu.sync_copy(dst_h.at[pl.ds(base0 + b * CH, CH)], didx.at[b])
        pltpu.make_async_copy(h_h.at[sidx.at[b]], rows.at[b], sems[b]).start()

    def body(i, carry):
        for b in range(2):
            pltpu.make_async_copy(h_h.at[sidx.at[b]], rows.at[b], sems[b]).wait()
            pltpu.sync_copy(rows.at[b], accum.at[didx.at[b]], add=True)

            @pl.when(i < NCH // 2 - 1)
            def _():
                nbase = base0 + (2 * i + b + 2) * CH
                pltpu.sync_copy(src_h.at[pl.ds(nbase, CH)], sidx.at[b])
                pltpu.sync_copy(dst_h.at[pl.ds(nbase, CH)], didx.at[b])
                pltpu.make_async_copy(h_h.at[sidx.at[b]], rows.at[b], sems[b]).start()

        return carry

    lax.fori_loop(0, NCH // 2, body, 0)
    plsc.subcore_barrier()
    pltpu.sync_copy(accum.at[pl.ds(s * RPT, RPT)], out_h.at[c, pl.ds(s * RPT, RPT)])


def _tc1_body(x_ref, do0, do1, di0, di1, hs_ref, ns_ref, nd_ref):
    deg_o = do0[0] + do1[0]
    deg_i = di0[0] + di1[0]
    ns = lax.rsqrt(jnp.maximum(deg_o, 1.0))
    nd = lax.rsqrt(jnp.maximum(deg_i, 1.0))
    ns_ref[...] = ns
    nd_ref[...] = nd
    hs_ref[...] = x_ref[...] * ns


def _tc1(x, deg_o_p, deg_i_p):
    return pl.pallas_call(
        _tc1_body,
        grid=(GRID,),
        in_specs=[
            pl.BlockSpec((BLK, D), lambda i: (i, 0)),
            pl.BlockSpec((1, BLK, 1), lambda i: (0, i, 0)),
            pl.BlockSpec((1, BLK, 1), lambda i: (1, i, 0)),
            pl.BlockSpec((1, BLK, 1), lambda i: (0, i, 0)),
            pl.BlockSpec((1, BLK, 1), lambda i: (1, i, 0)),
        ],
        out_specs=[
            pl.BlockSpec((BLK, D), lambda i: (i, 0)),
            pl.BlockSpec((BLK, 1), lambda i: (i, 0)),
            pl.BlockSpec((BLK, 1), lambda i: (i, 0)),
        ],
        out_shape=[
            jax.ShapeDtypeStruct((N, D), jnp.float32),
            jax.ShapeDtypeStruct((N, 1), jnp.float32),
            jax.ShapeDtypeStruct((N, 1), jnp.float32),
        ],
    )(x, deg_o_p, deg_o_p, deg_i_p, deg_i_p)


def _tc2_body(p_ref0, p_ref1, nd_ref, ns_ref, w_ref, b_ref, out_ref):
    m = (p_ref0[0] + p_ref1[0]) * nd_ref[...]
    h = jnp.dot(m, w_ref[...], preferred_element_type=jnp.float32) + b_ref[...]
    out_ref[...] = jnp.maximum(h, 0.0) * ns_ref[...]


def _tc2(m_p, nd, ns, w, b):
    return pl.pallas_call(
        _tc2_body,
        grid=(GRID,),
        in_specs=[
            pl.BlockSpec((1, BLK, D), lambda i: (0, i, 0)),
            pl.BlockSpec((1, BLK, D), lambda i: (1, i, 0)),
            pl.BlockSpec((BLK, 1), lambda i: (i, 0)),
            pl.BlockSpec((BLK, 1), lambda i: (i, 0)),
            pl.BlockSpec((D, D), lambda i: (0, 0)),
            pl.BlockSpec((1, D), lambda i: (0, 0)),
        ],
        out_specs=pl.BlockSpec((BLK, D), lambda i: (i, 0)),
        out_shape=jax.ShapeDtypeStruct((N, D), jnp.float32),
    )(m_p, m_p, nd, ns, w, b)


def _tc3_body(p_ref0, p_ref1, nd_ref, w_ref, b_ref, out_ref, acc_vec, acc_nrm):
    i = pl.program_id(0)
    m = (p_ref0[0] + p_ref1[0]) * nd_ref[...]
    h = jnp.dot(m, w_ref[...], preferred_element_type=jnp.float32) + b_ref[...]
    blk_sum = jnp.sum(h, axis=0, keepdims=True)
    blk_nrm = jnp.sum(jnp.sqrt(jnp.sum(h * h, axis=1)))

    @pl.when(i == 0)
    def _():
        acc_vec[...] = jnp.zeros_like(acc_vec)
        acc_nrm[0] = 0.0

    acc_vec[...] += blk_sum
    acc_nrm[0] += blk_nrm

    @pl.when(i == pl.num_programs(0) - 1)
    def _():
        factor = jnp.sqrt(jnp.float32(D)) * jnp.float32(N) / acc_nrm[0]
        out_ref[...] = acc_vec[...] * factor


def _tc3(m_p, nd, w, b):
    return pl.pallas_call(
        _tc3_body,
        grid=(GRID,),
        in_specs=[
            pl.BlockSpec((1, BLK, D), lambda i: (0, i, 0)),
            pl.BlockSpec((1, BLK, D), lambda i: (1, i, 0)),
            pl.BlockSpec((BLK, 1), lambda i: (i, 0)),
            pl.BlockSpec((D, D), lambda i: (0, 0)),
            pl.BlockSpec((1, D), lambda i: (0, 0)),
        ],
        out_specs=pl.BlockSpec((1, D), lambda i: (0, 0)),
        out_shape=jax.ShapeDtypeStruct((1, D), jnp.float32),
        scratch_shapes=[
            pltpu.VMEM((1, D), jnp.float32),
            pltpu.SMEM((1,), jnp.float32),
        ],
    )(m_p, m_p, nd, w, b)


def kernel(x, edge_index, W1, b1, W2, b2):
    ei = edge_index.astype(jnp.int32)
    src = ei[0]
    dst = ei[1]
    pad = E_PAD - E
    src_g = jnp.concatenate([src, jnp.zeros((pad,), jnp.int32)])
    src_d = jnp.concatenate([src, jnp.full((pad,), N, jnp.int32)])
    dst_p = jnp.concatenate([dst, jnp.full((pad,), N, jnp.int32)])

    zeros_d = jnp.zeros((N_PAD, D), jnp.float32)

    deg_o_f, deg_i_f = _deg_kernel(src_d, dst_p)
    deg_o_p = deg_o_f.reshape(NC, N_PAD, 1)
    deg_i_p = deg_i_f.reshape(NC, N_PAD, 1)
    h1s, ns, nd = _tc1(x, deg_o_p, deg_i_p)
    m1_p = _agg_kernel(h1s, src_g, dst_p, zeros_d)
    h2s = _tc2(m1_p, nd, ns, W1, b1.reshape(1, D))
    m2_p = _agg_kernel(h2s, src_g, dst_p, zeros_d)
    out = _tc3(m2_p, nd, W2, b2.reshape(1, D))
    return out


# speedup vs baseline: 2.8241x; 1.0030x over previous
import functools

import jax
import jax.numpy as jnp
from jax import lax
from jax.experimental import pallas as pl
from jax.experimental.pallas import tpu as pltpu
from jax.experimental.pallas import tpu_sc as plsc

N = 10000
D = 128
E = 320000

NC = 2
NS = 16
NW = NC * NS
CH = 128
EPW = 10240
E_PAD = EPW * NW
NCH = EPW // CH
CHA = 64
NCHA = EPW // CHA
N_PAD = 10240
RPT = N_PAD // NS

BLK = 1000
GRID = N // BLK

_mesh = plsc.VectorSubcoreMesh(core_axis_name="c", subcore_axis_name="s")


@functools.partial(
    pl.kernel,
    mesh=_mesh,
    out_type=(
        jax.ShapeDtypeStruct((NC * N_PAD,), jnp.float32),
        jax.ShapeDtypeStruct((NC * N_PAD,), jnp.float32),
    ),
    scratch_types=[
        pltpu.VMEM((CH,), jnp.int32),
        pltpu.VMEM((CH,), jnp.int32),
        pltpu.VMEM((CH,), jnp.float32),
        pltpu.VMEM((CH,), jnp.float32),
        pltpu.VMEM_SHARED((N_PAD,), jnp.float32),
        pltpu.VMEM_SHARED((N_PAD,), jnp.float32),
    ],
)
def _deg_kernel(src_h, dst_h, deg_o_h, deg_i_h,
                sidx, didx, ones_v, zeros_v, deg_o_s, deg_i_s):
    c = lax.axis_index("c")
    s = lax.axis_index("s")
    wid = s * NC + c
    for i in range(CH // 16):
        ones_v[pl.ds(i * 16, 16)] = jnp.ones((16,), jnp.float32)
        zeros_v[pl.ds(i * 16, 16)] = jnp.zeros((16,), jnp.float32)
    for j in range(RPT // CH):
        pltpu.sync_copy(zeros_v, deg_o_s.at[pl.ds(s * RPT + j * CH, CH)])
        pltpu.sync_copy(zeros_v, deg_i_s.at[pl.ds(s * RPT + j * CH, CH)])
    plsc.subcore_barrier()

    base0 = wid * EPW

    def body(i, carry):
        base = base0 + i * CH
        pltpu.sync_copy(src_h.at[pl.ds(base, CH)], sidx)
        pltpu.sync_copy(dst_h.at[pl.ds(base, CH)], didx)
        pltpu.sync_copy(ones_v, deg_o_s.at[sidx], add=True)
        pltpu.sync_copy(ones_v, deg_i_s.at[didx], add=True)
        return carry

    lax.fori_loop(0, NCH, body, 0)
    plsc.subcore_barrier()
    pltpu.sync_copy(deg_o_s.at[pl.ds(s * RPT, RPT)],
                    deg_o_h.at[pl.ds(c * N_PAD + s * RPT, RPT)])
    pltpu.sync_copy(deg_i_s.at[pl.ds(s * RPT, RPT)],
                    deg_i_h.at[pl.ds(c * N_PAD + s * RPT, RPT)])


@functools.partial(
    pl.kernel,
    mesh=_mesh,
    out_type=jax.ShapeDtypeStruct((NC, N_PAD, D), jnp.float32),
    scratch_types=[
        pltpu.VMEM((4, CHA), jnp.int32),
        pltpu.VMEM((4, CHA), jnp.int32),
        pltpu.VMEM((4, CHA, D), jnp.float32),
        pltpu.VMEM_SHARED((N_PAD, D), jnp.float32),
        pltpu.SemaphoreType.DMA,
        pltpu.SemaphoreType.DMA,
        pltpu.SemaphoreType.DMA,
        pltpu.SemaphoreType.DMA,
        pltpu.SemaphoreType.DMA,
        pltpu.SemaphoreType.DMA,
        pltpu.SemaphoreType.DMA,
        pltpu.SemaphoreType.DMA,
    ],
)
def _agg_kernel(h_h, src_h, dst_h, zeros_h, out_h, sidx, didx, rows, accum,
                g0, g1, g2, g3, s0, s1, s2, s3):
    c = lax.axis_index("c")
    s = lax.axis_index("s")
    wid = s * NC + c
    pltpu.sync_copy(zeros_h.at[pl.ds(s * RPT, RPT)], accum.at[pl.ds(s * RPT, RPT)])
    plsc.subcore_barrier()

    base0 = wid * EPW
    gsems = (g0, g1, g2, g3)
    ssems = (s0, s1, s2, s3)

    def load_and_gather(chunk, b):
        base = base0 + chunk * CHA
        pltpu.sync_copy(src_h.at[pl.ds(base, CHA)], sidx.at[b])
        pltpu.sync_copy(dst_h.at[pl.ds(base, CHA)], didx.at[b])
        pltpu.async_copy(h_h.at[sidx.at[b]], rows.at[b], gsems[b])

    for b in range(2):
        load_and_gather(b, b)

    def body(i, carry):
        for b in range(4):
            ch = 4 * i + b
            pltpu.make_async_copy(h_h.at[sidx.at[b]], rows.at[b], gsems[b]).wait()
            pltpu.async_copy(rows.at[b], accum.at[didx.at[b]], ssems[b], add=True)
            b2 = (b + 2) % 4

            @pl.when(ch + 2 < NCHA)
            def _():
                @pl.when(ch >= 2)
                def _():
                    pltpu.make_async_copy(
                        rows.at[b2], accum.at[didx.at[b2]], ssems[b2]).wait()

                load_and_gather(ch + 2, b2)

        return carry

    lax.fori_loop(0, NCHA // 4, body, 0)
    for b in range(4):
        pltpu.make_async_copy(rows.at[b], accum.at[didx.at[b]], ssems[b]).wait()
    plsc.subcore_barrier()
    pltpu.sync_copy(accum.at[pl.ds(s * RPT, RPT)], out_h.at[c, pl.ds(s * RPT, RPT)])


def _tc1_body(x_ref, do0, do1, di0, di1, hs_ref, ns_ref, nd_ref):
    deg_o = do0[0] + do1[0]
    deg_i = di0[0] + di1[0]
    ns = lax.rsqrt(jnp.maximum(deg_o, 1.0))
    nd = lax.rsqrt(jnp.maximum(deg_i, 1.0))
    ns_ref[...] = ns
    nd_ref[...] = nd
    hs_ref[...] = x_ref[...] * ns


def _tc1(x, deg_o_p, deg_i_p):
    return pl.pallas_call(
        _tc1_body,
        grid=(GRID,),
        in_specs=[
            pl.BlockSpec((BLK, D), lambda i: (i, 0)),
            pl.BlockSpec((1, BLK, 1), lambda i: (0, i, 0)),
            pl.BlockSpec((1, BLK, 1), lambda i: (1, i, 0)),
            pl.BlockSpec((1, BLK, 1), lambda i: (0, i, 0)),
            pl.BlockSpec((1, BLK, 1), lambda i: (1, i, 0)),
        ],
        out_specs=[
            pl.BlockSpec((BLK, D), lambda i: (i, 0)),
            pl.BlockSpec((BLK, 1), lambda i: (i, 0)),
            pl.BlockSpec((BLK, 1), lambda i: (i, 0)),
        ],
        out_shape=[
            jax.ShapeDtypeStruct((N, D), jnp.float32),
            jax.ShapeDtypeStruct((N, 1), jnp.float32),
            jax.ShapeDtypeStruct((N, 1), jnp.float32),
        ],
    )(x, deg_o_p, deg_o_p, deg_i_p, deg_i_p)


def _tc2_body(p_ref0, p_ref1, nd_ref, ns_ref, w_ref, b_ref, out_ref):
    m = (p_ref0[0] + p_ref1[0]) * nd_ref[...]
    h = jnp.dot(m, w_ref[...], preferred_element_type=jnp.float32) + b_ref[...]
    out_ref[...] = jnp.maximum(h, 0.0) * ns_ref[...]


def _tc2(m_p, nd, ns, w, b):
    return pl.pallas_call(
        _tc2_body,
        grid=(GRID,),
        in_specs=[
            pl.BlockSpec((1, BLK, D), lambda i: (0, i, 0)),
            pl.BlockSpec((1, BLK, D), lambda i: (1, i, 0)),
            pl.BlockSpec((BLK, 1), lambda i: (i, 0)),
            pl.BlockSpec((BLK, 1), lambda i: (i, 0)),
            pl.BlockSpec((D, D), lambda i: (0, 0)),
            pl.BlockSpec((1, D), lambda i: (0, 0)),
        ],
        out_specs=pl.BlockSpec((BLK, D), lambda i: (i, 0)),
        out_shape=jax.ShapeDtypeStruct((N, D), jnp.float32),
    )(m_p, m_p, nd, ns, w, b)


def _tc3_body(p_ref0, p_ref1, nd_ref, w_ref, b_ref, out_ref, acc_vec, acc_nrm):
    i = pl.program_id(0)
    m = (p_ref0[0] + p_ref1[0]) * nd_ref[...]
    h = jnp.dot(m, w_ref[...], preferred_element_type=jnp.float32) + b_ref[...]
    blk_sum = jnp.sum(h, axis=0, keepdims=True)
    blk_nrm = jnp.sum(jnp.sqrt(jnp.sum(h * h, axis=1)))

    @pl.when(i == 0)
    def _():
        acc_vec[...] = jnp.zeros_like(acc_vec)
        acc_nrm[0] = 0.0

    acc_vec[...] += blk_sum
    acc_nrm[0] += blk_nrm

    @pl.when(i == pl.num_programs(0) - 1)
    def _():
        factor = jnp.sqrt(jnp.float32(D)) * jnp.float32(N) / acc_nrm[0]
        out_ref[...] = acc_vec[...] * factor


def _tc3(m_p, nd, w, b):
    return pl.pallas_call(
        _tc3_body,
        grid=(GRID,),
        in_specs=[
            pl.BlockSpec((1, BLK, D), lambda i: (0, i, 0)),
            pl.BlockSpec((1, BLK, D), lambda i: (1, i, 0)),
            pl.BlockSpec((BLK, 1), lambda i: (i, 0)),
            pl.BlockSpec((D, D), lambda i: (0, 0)),
            pl.BlockSpec((1, D), lambda i: (0, 0)),
        ],
        out_specs=pl.BlockSpec((1, D), lambda i: (0, 0)),
        out_shape=jax.ShapeDtypeStruct((1, D), jnp.float32),
        scratch_shapes=[
            pltpu.VMEM((1, D), jnp.float32),
            pltpu.SMEM((1,), jnp.float32),
        ],
    )(m_p, m_p, nd, w, b)


def kernel(x, edge_index, W1, b1, W2, b2):
    ei = edge_index.astype(jnp.int32)
    src = ei[0]
    dst = ei[1]
    pad = E_PAD - E
    src_g = jnp.concatenate([src, jnp.zeros((pad,), jnp.int32)])
    src_d = jnp.concatenate([src, jnp.full((pad,), N, jnp.int32)])
    dst_p = jnp.concatenate([dst, jnp.full((pad,), N, jnp.int32)])

    zeros_d = jnp.zeros((N_PAD, D), jnp.float32)

    deg_o_f, deg_i_f = _deg_kernel(src_d, dst_p)
    deg_o_p = deg_o_f.reshape(NC, N_PAD, 1)
    deg_i_p = deg_i_f.reshape(NC, N_PAD, 1)
    h1s, ns, nd = _tc1(x, deg_o_p, deg_i_p)
    m1_p = _agg_kernel(h1s, src_g, dst_p, zeros_d)
    h2s = _tc2(m1_p, nd, ns, W1, b1.reshape(1, D))
    m2_p = _agg_kernel(h2s, src_g, dst_p, zeros_d)
    out = _tc3(m2_p, nd, W2, b2.reshape(1, D))
    return out


# speedup vs baseline: 3.0175x; 1.0685x over previous
import functools

import jax
import jax.numpy as jnp
from jax import lax
from jax.experimental import pallas as pl
from jax.experimental.pallas import tpu as pltpu
from jax.experimental.pallas import tpu_sc as plsc

N = 10000
D = 128
E = 320000

NC = 2
NS = 16
NW = NC * NS
CH = 128
EPW = 10240
E_PAD = EPW * NW
NCH = EPW // CH
CHA = 64
NCHA = EPW // CHA
EPW0 = 14336
EPW1 = 6144
OFF1 = NS * EPW0
N_PAD = 10240
RPT = N_PAD // NS

BLK = 1000
GRID = N // BLK

_mesh = plsc.VectorSubcoreMesh(core_axis_name="c", subcore_axis_name="s")


@functools.partial(
    pl.kernel,
    mesh=_mesh,
    out_type=(
        jax.ShapeDtypeStruct((NC * N_PAD,), jnp.float32),
        jax.ShapeDtypeStruct((NC * N_PAD,), jnp.float32),
    ),
    scratch_types=[
        pltpu.VMEM((CH,), jnp.int32),
        pltpu.VMEM((CH,), jnp.int32),
        pltpu.VMEM((CH,), jnp.float32),
        pltpu.VMEM((CH,), jnp.float32),
        pltpu.VMEM_SHARED((N_PAD,), jnp.float32),
        pltpu.VMEM_SHARED((N_PAD,), jnp.float32),
    ],
)
def _deg_kernel(src_h, dst_h, deg_o_h, deg_i_h,
                sidx, didx, ones_v, zeros_v, deg_o_s, deg_i_s):
    c = lax.axis_index("c")
    s = lax.axis_index("s")
    wid = s * NC + c
    for i in range(CH // 16):
        ones_v[pl.ds(i * 16, 16)] = jnp.ones((16,), jnp.float32)
        zeros_v[pl.ds(i * 16, 16)] = jnp.zeros((16,), jnp.float32)
    for j in range(RPT // CH):
        pltpu.sync_copy(zeros_v, deg_o_s.at[pl.ds(s * RPT + j * CH, CH)])
        pltpu.sync_copy(zeros_v, deg_i_s.at[pl.ds(s * RPT + j * CH, CH)])
    plsc.subcore_barrier()

    base0 = wid * EPW

    def body(i, carry):
        base = base0 + i * CH
        pltpu.sync_copy(src_h.at[pl.ds(base, CH)], sidx)
        pltpu.sync_copy(dst_h.at[pl.ds(base, CH)], didx)
        pltpu.sync_copy(ones_v, deg_o_s.at[sidx], add=True)
        pltpu.sync_copy(ones_v, deg_i_s.at[didx], add=True)
        return carry

    lax.fori_loop(0, NCH, body, 0)
    plsc.subcore_barrier()
    pltpu.sync_copy(deg_o_s.at[pl.ds(s * RPT, RPT)],
                    deg_o_h.at[pl.ds(c * N_PAD + s * RPT, RPT)])
    pltpu.sync_copy(deg_i_s.at[pl.ds(s * RPT, RPT)],
                    deg_i_h.at[pl.ds(c * N_PAD + s * RPT, RPT)])


@functools.partial(
    pl.kernel,
    mesh=_mesh,
    out_type=jax.ShapeDtypeStruct((NC, N_PAD, D), jnp.float32),
    scratch_types=[
        pltpu.VMEM((4, CHA), jnp.int32),
        pltpu.VMEM((4, CHA), jnp.int32),
        pltpu.VMEM((4, CHA, D), jnp.float32),
        pltpu.VMEM_SHARED((N_PAD, D), jnp.float32),
        pltpu.SemaphoreType.DMA,
        pltpu.SemaphoreType.DMA,
        pltpu.SemaphoreType.DMA,
        pltpu.SemaphoreType.DMA,
        pltpu.SemaphoreType.DMA,
        pltpu.SemaphoreType.DMA,
        pltpu.SemaphoreType.DMA,
        pltpu.SemaphoreType.DMA,
    ],
)
def _agg_kernel(h_h, src_h, dst_h, zeros_h, out_h, sidx, didx, rows, accum,
                g0, g1, g2, g3, s0, s1, s2, s3):
    c = lax.axis_index("c")
    s = lax.axis_index("s")
    wid = s * NC + c
    pltpu.sync_copy(zeros_h.at[pl.ds(s * RPT, RPT)], accum.at[pl.ds(s * RPT, RPT)])
    plsc.subcore_barrier()

    is0 = c == 0
    base0 = jnp.where(is0, s * EPW0, OFF1 + s * EPW1)
    nch = jnp.where(is0, EPW0 // CHA, EPW1 // CHA)
    gsems = (g0, g1, g2, g3)
    ssems = (s0, s1, s2, s3)

    def load_and_gather(chunk, b):
        base = base0 + chunk * CHA
        pltpu.sync_copy(src_h.at[pl.ds(base, CHA)], sidx.at[b])
        pltpu.sync_copy(dst_h.at[pl.ds(base, CHA)], didx.at[b])
        pltpu.async_copy(h_h.at[sidx.at[b]], rows.at[b], gsems[b])

    for b in range(2):
        load_and_gather(b, b)

    def body(i, carry):
        for b in range(4):
            ch = 4 * i + b
            pltpu.make_async_copy(h_h.at[sidx.at[b]], rows.at[b], gsems[b]).wait()
            pltpu.async_copy(rows.at[b], accum.at[didx.at[b]], ssems[b], add=True)
            b2 = (b + 2) % 4

            @pl.when(ch + 2 < nch)
            def _():
                @pl.when(ch >= 2)
                def _():
                    pltpu.make_async_copy(
                        rows.at[b2], accum.at[didx.at[b2]], ssems[b2]).wait()

                load_and_gather(ch + 2, b2)

        return carry

    lax.fori_loop(0, nch // 4, body, 0)
    for b in range(4):
        pltpu.make_async_copy(rows.at[b], accum.at[didx.at[b]], ssems[b]).wait()
    plsc.subcore_barrier()
    pltpu.sync_copy(accum.at[pl.ds(s * RPT, RPT)], out_h.at[c, pl.ds(s * RPT, RPT)])


def _tc1_body(x_ref, do0, do1, di0, di1, hs_ref, ns_ref, nd_ref):
    deg_o = do0[0] + do1[0]
    deg_i = di0[0] + di1[0]
    ns = lax.rsqrt(jnp.maximum(deg_o, 1.0))
    nd = lax.rsqrt(jnp.maximum(deg_i, 1.0))
    ns_ref[...] = ns
    nd_ref[...] = nd
    hs_ref[...] = x_ref[...] * ns


def _tc1(x, deg_o_p, deg_i_p):
    return pl.pallas_call(
        _tc1_body,
        grid=(GRID,),
        in_specs=[
            pl.BlockSpec((BLK, D), lambda i: (i, 0)),
            pl.BlockSpec((1, BLK, 1), lambda i: (0, i, 0)),
            pl.BlockSpec((1, BLK, 1), lambda i: (1, i, 0)),
            pl.BlockSpec((1, BLK, 1), lambda i: (0, i, 0)),
            pl.BlockSpec((1, BLK, 1), lambda i: (1, i, 0)),
        ],
        out_specs=[
            pl.BlockSpec((BLK, D), lambda i: (i, 0)),
            pl.BlockSpec((BLK, 1), lambda i: (i, 0)),
            pl.BlockSpec((BLK, 1), lambda i: (i, 0)),
        ],
        out_shape=[
            jax.ShapeDtypeStruct((N, D), jnp.float32),
            jax.ShapeDtypeStruct((N, 1), jnp.float32),
            jax.ShapeDtypeStruct((N, 1), jnp.float32),
        ],
    )(x, deg_o_p, deg_o_p, deg_i_p, deg_i_p)


def _tc2_body(p_ref0, p_ref1, nd_ref, ns_ref, w_ref, b_ref, out_ref):
    m = (p_ref0[0] + p_ref1[0]) * nd_ref[...]
    h = jnp.dot(m, w_ref[...], preferred_element_type=jnp.float32) + b_ref[...]
    out_ref[...] = jnp.maximum(h, 0.0) * ns_ref[...]


def _tc2(m_p, nd, ns, w, b):
    return pl.pallas_call(
        _tc2_body,
        grid=(GRID,),
        in_specs=[
            pl.BlockSpec((1, BLK, D), lambda i: (0, i, 0)),
            pl.BlockSpec((1, BLK, D), lambda i: (1, i, 0)),
            pl.BlockSpec((BLK, 1), lambda i: (i, 0)),
            pl.BlockSpec((BLK, 1), lambda i: (i, 0)),
            pl.BlockSpec((D, D), lambda i: (0, 0)),
            pl.BlockSpec((1, D), lambda i: (0, 0)),
        ],
        out_specs=pl.BlockSpec((BLK, D), lambda i: (i, 0)),
        out_shape=jax.ShapeDtypeStruct((N, D), jnp.float32),
    )(m_p, m_p, nd, ns, w, b)


def _tc3_body(p_ref0, p_ref1, nd_ref, w_ref, b_ref, out_ref, acc_vec, acc_nrm):
    i = pl.program_id(0)
    m = (p_ref0[0] + p_ref1[0]) * nd_ref[...]
    h = jnp.dot(m, w_ref[...], preferred_element_type=jnp.float32) + b_ref[...]
    blk_sum = jnp.sum(h, axis=0, keepdims=True)
    blk_nrm = jnp.sum(jnp.sqrt(jnp.sum(h * h, axis=1)))

    @pl.when(i == 0)
    def _():
        acc_vec[...] = jnp.zeros_like(acc_vec)
        acc_nrm[0] = 0.0

    acc_vec[...] += blk_sum
    acc_nrm[0] += blk_nrm

    @pl.when(i == pl.num_programs(0) - 1)
    def _():
        factor = jnp.sqrt(jnp.float32(D)) * jnp.float32(N) / acc_nrm[0]
        out_ref[...] = acc_vec[...] * factor


def _tc3(m_p, nd, w, b):
    return pl.pallas_call(
        _tc3_body,
        grid=(GRID,),
        in_specs=[
            pl.BlockSpec((1, BLK, D), lambda i: (0, i, 0)),
            pl.BlockSpec((1, BLK, D), lambda i: (1, i, 0)),
            pl.BlockSpec((BLK, 1), lambda i: (i, 0)),
            pl.BlockSpec((D, D), lambda i: (0, 0)),
            pl.BlockSpec((1, D), lambda i: (0, 0)),
        ],
        out_specs=pl.BlockSpec((1, D), lambda i: (0, 0)),
        out_shape=jax.ShapeDtypeStruct((1, D), jnp.float32),
        scratch_shapes=[
            pltpu.VMEM((1, D), jnp.float32),
            pltpu.SMEM((1,), jnp.float32),
        ],
    )(m_p, m_p, nd, w, b)


def kernel(x, edge_index, W1, b1, W2, b2):
    ei = edge_index.astype(jnp.int32)
    src = ei[0]
    dst = ei[1]
    pad = E_PAD - E
    src_g = jnp.concatenate([src, jnp.zeros((pad,), jnp.int32)])
    src_d = jnp.concatenate([src, jnp.full((pad,), N, jnp.int32)])
    dst_p = jnp.concatenate([dst, jnp.full((pad,), N, jnp.int32)])

    zeros_d = jnp.zeros((N_PAD, D), jnp.float32)

    deg_o_f, deg_i_f = _deg_kernel(src_d, dst_p)
    deg_o_p = deg_o_f.reshape(NC, N_PAD, 1)
    deg_i_p = deg_i_f.reshape(NC, N_PAD, 1)
    h1s, ns, nd = _tc1(x, deg_o_p, deg_i_p)
    m1_p = _agg_kernel(h1s, src_g, dst_p, zeros_d)
    h2s = _tc2(m1_p, nd, ns, W1, b1.reshape(1, D))
    m2_p = _agg_kernel(h2s, src_g, dst_p, zeros_d)
    out = _tc3(m2_p, nd, W2, b2.reshape(1, D))
    return out


# speedup vs baseline: 3.0271x; 1.0032x over previous
import functools

import jax
import jax.numpy as jnp
from jax import lax
from jax.experimental import pallas as pl
from jax.experimental.pallas import tpu as pltpu
from jax.experimental.pallas import tpu_sc as plsc

N = 10000
D = 128
E = 320000

NC = 2
NS = 16
NW = NC * NS
CH = 128
EPW = 10240
E_PAD = EPW * NW
NCH = EPW // CH
CHA = 64
NCHA = EPW // CHA
EPW0 = 15360
EPW1 = 5120
OFF1 = NS * EPW0
N_PAD = 10240
RPT = N_PAD // NS

BLK = 1000
GRID = N // BLK

_mesh = plsc.VectorSubcoreMesh(core_axis_name="c", subcore_axis_name="s")


@functools.partial(
    pl.kernel,
    mesh=_mesh,
    out_type=(
        jax.ShapeDtypeStruct((NC * N_PAD,), jnp.float32),
        jax.ShapeDtypeStruct((NC * N_PAD,), jnp.float32),
    ),
    scratch_types=[
        pltpu.VMEM((CH,), jnp.int32),
        pltpu.VMEM((CH,), jnp.int32),
        pltpu.VMEM((CH,), jnp.float32),
        pltpu.VMEM((CH,), jnp.float32),
        pltpu.VMEM_SHARED((N_PAD,), jnp.float32),
        pltpu.VMEM_SHARED((N_PAD,), jnp.float32),
    ],
)
def _deg_kernel(src_h, dst_h, deg_o_h, deg_i_h,
                sidx, didx, ones_v, zeros_v, deg_o_s, deg_i_s):
    c = lax.axis_index("c")
    s = lax.axis_index("s")
    wid = s * NC + c
    for i in range(CH // 16):
        ones_v[pl.ds(i * 16, 16)] = jnp.ones((16,), jnp.float32)
        zeros_v[pl.ds(i * 16, 16)] = jnp.zeros((16,), jnp.float32)
    for j in range(RPT // CH):
        pltpu.sync_copy(zeros_v, deg_o_s.at[pl.ds(s * RPT + j * CH, CH)])
        pltpu.sync_copy(zeros_v, deg_i_s.at[pl.ds(s * RPT + j * CH, CH)])
    plsc.subcore_barrier()

    base0 = wid * EPW

    def body(i, carry):
        base = base0 + i * CH
        pltpu.sync_copy(src_h.at[pl.ds(base, CH)], sidx)
        pltpu.sync_copy(dst_h.at[pl.ds(base, CH)], didx)
        pltpu.sync_copy(ones_v, deg_o_s.at[sidx], add=True)
        pltpu.sync_copy(ones_v, deg_i_s.at[didx], add=True)
        return carry

    lax.fori_loop(0, NCH, body, 0)
    plsc.subcore_barrier()
    pltpu.sync_copy(deg_o_s.at[pl.ds(s * RPT, RPT)],
                    deg_o_h.at[pl.ds(c * N_PAD + s * RPT, RPT)])
    pltpu.sync_copy(deg_i_s.at[pl.ds(s * RPT, RPT)],
                    deg_i_h.at[pl.ds(c * N_PAD + s * RPT, RPT)])


@functools.partial(
    pl.kernel,
    mesh=_mesh,
    out_type=jax.ShapeDtypeStruct((NC, N_PAD, D), jnp.float32),
    scratch_types=[
        pltpu.VMEM((4, CHA), jnp.int32),
        pltpu.VMEM((4, CHA), jnp.int32),
        pltpu.VMEM((4, CHA, D), jnp.float32),
        pltpu.VMEM_SHARED((N_PAD, D), jnp.float32),
        pltpu.SemaphoreType.DMA,
        pltpu.SemaphoreType.DMA,
        pltpu.SemaphoreType.DMA,
        pltpu.SemaphoreType.DMA,
        pltpu.SemaphoreType.DMA,
        pltpu.SemaphoreType.DMA,
        pltpu.SemaphoreType.DMA,
        pltpu.SemaphoreType.DMA,
    ],
)
def _agg_kernel(h_h, src_h, dst_h, zeros_h, out_h, sidx, didx, rows, accum,
                g0, g1, g2, g3, s0, s1, s2, s3):
    c = lax.axis_index("c")
    s = lax.axis_index("s")
    wid = s * NC + c
    pltpu.sync_copy(zeros_h.at[pl.ds(s * RPT, RPT)], accum.at[pl.ds(s * RPT, RPT)])
    plsc.subcore_barrier()

    is0 = c == 0
    base0 = jnp.where(is0, s * EPW0, OFF1 + s * EPW1)
    nch = jnp.where(is0, EPW0 // CHA, EPW1 // CHA)
    gsems = (g0, g1, g2, g3)
    ssems = (s0, s1, s2, s3)

    def load_and_gather(chunk, b):
        base = base0 + chunk * CHA
        pltpu.sync_copy(src_h.at[pl.ds(base, CHA)], sidx.at[b])
        pltpu.sync_copy(dst_h.at[pl.ds(base, CHA)], didx.at[b])
        pltpu.async_copy(h_h.at[sidx.at[b]], rows.at[b], gsems[b])

    for b in range(2):
        load_and_gather(b, b)

    def body(i, carry):
        for b in range(4):
            ch = 4 * i + b
            pltpu.make_async_copy(h_h.at[sidx.at[b]], rows.at[b], gsems[b]).wait()
            pltpu.async_copy(rows.at[b], accum.at[didx.at[b]], ssems[b], add=True)
            b2 = (b + 2) % 4

            @pl.when(ch + 2 < nch)
            def _():
                @pl.when(ch >= 2)
                def _():
                    pltpu.make_async_copy(
                        rows.at[b2], accum.at[didx.at[b2]], ssems[b2]).wait()

                load_and_gather(ch + 2, b2)

        return carry

    lax.fori_loop(0, nch // 4, body, 0)
    for b in range(4):
        pltpu.make_async_copy(rows.at[b], accum.at[didx.at[b]], ssems[b]).wait()
    plsc.subcore_barrier()
    pltpu.sync_copy(accum.at[pl.ds(s * RPT, RPT)], out_h.at[c, pl.ds(s * RPT, RPT)])


def _tc1_body(x_ref, do0, do1, di0, di1, hs_ref, ns_ref, nd_ref):
    deg_o = do0[0] + do1[0]
    deg_i = di0[0] + di1[0]
    ns = lax.rsqrt(jnp.maximum(deg_o, 1.0))
    nd = lax.rsqrt(jnp.maximum(deg_i, 1.0))
    ns_ref[...] = ns
    nd_ref[...] = nd
    hs_ref[...] = x_ref[...] * ns


def _tc1(x, deg_o_p, deg_i_p):
    return pl.pallas_call(
        _tc1_body,
        grid=(GRID,),
        in_specs=[
            pl.BlockSpec((BLK, D), lambda i: (i, 0)),
            pl.BlockSpec((1, BLK, 1), lambda i: (0, i, 0)),
            pl.BlockSpec((1, BLK, 1), lambda i: (1, i, 0)),
            pl.BlockSpec((1, BLK, 1), lambda i: (0, i, 0)),
            pl.BlockSpec((1, BLK, 1), lambda i: (1, i, 0)),
        ],
        out_specs=[
            pl.BlockSpec((BLK, D), lambda i: (i, 0)),
            pl.BlockSpec((BLK, 1), lambda i: (i, 0)),
            pl.BlockSpec((BLK, 1), lambda i: (i, 0)),
        ],
        out_shape=[
            jax.ShapeDtypeStruct((N, D), jnp.float32),
            jax.ShapeDtypeStruct((N, 1), jnp.float32),
            jax.ShapeDtypeStruct((N, 1), jnp.float32),
        ],
    )(x, deg_o_p, deg_o_p, deg_i_p, deg_i_p)


def _tc2_body(p_ref0, p_ref1, nd_ref, ns_ref, w_ref, b_ref, out_ref):
    m = (p_ref0[0] + p_ref1[0]) * nd_ref[...]
    h = jnp.dot(m, w_ref[...], preferred_element_type=jnp.float32) + b_ref[...]
    out_ref[...] = jnp.maximum(h, 0.0) * ns_ref[...]


def _tc2(m_p, nd, ns, w, b):
    return pl.pallas_call(
        _tc2_body,
        grid=(GRID,),
        in_specs=[
            pl.BlockSpec((1, BLK, D), lambda i: (0, i, 0)),
            pl.BlockSpec((1, BLK, D), lambda i: (1, i, 0)),
            pl.BlockSpec((BLK, 1), lambda i: (i, 0)),
            pl.BlockSpec((BLK, 1), lambda i: (i, 0)),
            pl.BlockSpec((D, D), lambda i: (0, 0)),
            pl.BlockSpec((1, D), lambda i: (0, 0)),
        ],
        out_specs=pl.BlockSpec((BLK, D), lambda i: (i, 0)),
        out_shape=jax.ShapeDtypeStruct((N, D), jnp.float32),
    )(m_p, m_p, nd, ns, w, b)


def _tc3_body(p_ref0, p_ref1, nd_ref, w_ref, b_ref, out_ref, acc_vec, acc_nrm):
    i = pl.program_id(0)
    m = (p_ref0[0] + p_ref1[0]) * nd_ref[...]
    h = jnp.dot(m, w_ref[...], preferred_element_type=jnp.float32) + b_ref[...]
    blk_sum = jnp.sum(h, axis=0, keepdims=True)
    blk_nrm = jnp.sum(jnp.sqrt(jnp.sum(h * h, axis=1)))

    @pl.when(i == 0)
    def _():
        acc_vec[...] = jnp.zeros_like(acc_vec)
        acc_nrm[0] = 0.0

    acc_vec[...] += blk_sum
    acc_nrm[0] += blk_nrm

    @pl.when(i == pl.num_programs(0) - 1)
    def _():
        factor = jnp.sqrt(jnp.float32(D)) * jnp.float32(N) / acc_nrm[0]
        out_ref[...] = acc_vec[...] * factor


def _tc3(m_p, nd, w, b):
    return pl.pallas_call(
        _tc3_body,
        grid=(GRID,),
        in_specs=[
            pl.BlockSpec((1, BLK, D), lambda i: (0, i, 0)),
            pl.BlockSpec((1, BLK, D), lambda i: (1, i, 0)),
            pl.BlockSpec((BLK, 1), lambda i: (i, 0)),
            pl.BlockSpec((D, D), lambda i: (0, 0)),
            pl.BlockSpec((1, D), lambda i: (0, 0)),
        ],
        out_specs=pl.BlockSpec((1, D), lambda i: (0, 0)),
        out_shape=jax.ShapeDtypeStruct((1, D), jnp.float32),
        scratch_shapes=[
            pltpu.VMEM((1, D), jnp.float32),
            pltpu.SMEM((1,), jnp.float32),
        ],
    )(m_p, m_p, nd, w, b)


def kernel(x, edge_index, W1, b1, W2, b2):
    ei = edge_index.astype(jnp.int32)
    src = ei[0]
    dst = ei[1]
    pad = E_PAD - E
    src_g = jnp.concatenate([src, jnp.zeros((pad,), jnp.int32)])
    src_d = jnp.concatenate([src, jnp.full((pad,), N, jnp.int32)])
    dst_p = jnp.concatenate([dst, jnp.full((pad,), N, jnp.int32)])

    zeros_d = jnp.zeros((N_PAD, D), jnp.float32)

    deg_o_f, deg_i_f = _deg_kernel(src_d, dst_p)
    deg_o_p = deg_o_f.reshape(NC, N_PAD, 1)
    deg_i_p = deg_i_f.reshape(NC, N_PAD, 1)
    h1s, ns, nd = _tc1(x, deg_o_p, deg_i_p)
    m1_p = _agg_kernel(h1s, src_g, dst_p, zeros_d)
    h2s = _tc2(m1_p, nd, ns, W1, b1.reshape(1, D))
    m2_p = _agg_kernel(h2s, src_g, dst_p, zeros_d)
    out = _tc3(m2_p, nd, W2, b2.reshape(1, D))
    return out


# speedup vs baseline: 3.2214x; 1.0642x over previous
import functools

import jax
import jax.numpy as jnp
from jax import lax
from jax.experimental import pallas as pl
from jax.experimental.pallas import tpu as pltpu
from jax.experimental.pallas import tpu_sc as plsc

N = 10000
D = 128
E = 320000

NC = 2
NS = 16
NW = NC * NS
CH = 128
EPW = 10240
E_PAD = EPW * NW
NCH = EPW // CH
CHA = 64
NCHA = EPW // CHA
EPW0 = 15360
EPW1 = 5120
OFF1 = NS * EPW0
N_PAD = 10240
RPT = N_PAD // NS

BLK = 1000
GRID = N // BLK

_mesh = plsc.VectorSubcoreMesh(core_axis_name="c", subcore_axis_name="s")


@functools.partial(
    pl.kernel,
    mesh=_mesh,
    out_type=(
        jax.ShapeDtypeStruct((NC * N_PAD,), jnp.float32),
        jax.ShapeDtypeStruct((NC * N_PAD,), jnp.float32),
    ),
    scratch_types=[
        pltpu.VMEM((4, CH), jnp.int32),
        pltpu.VMEM((4, CH), jnp.int32),
        pltpu.VMEM((CH,), jnp.float32),
        pltpu.VMEM((CH,), jnp.float32),
        pltpu.VMEM_SHARED((N_PAD,), jnp.float32),
        pltpu.VMEM_SHARED((N_PAD,), jnp.float32),
        pltpu.SemaphoreType.DMA,
        pltpu.SemaphoreType.DMA,
        pltpu.SemaphoreType.DMA,
        pltpu.SemaphoreType.DMA,
        pltpu.SemaphoreType.DMA,
        pltpu.SemaphoreType.DMA,
        pltpu.SemaphoreType.DMA,
        pltpu.SemaphoreType.DMA,
        pltpu.SemaphoreType.DMA,
        pltpu.SemaphoreType.DMA,
        pltpu.SemaphoreType.DMA,
        pltpu.SemaphoreType.DMA,
        pltpu.SemaphoreType.DMA,
        pltpu.SemaphoreType.DMA,
        pltpu.SemaphoreType.DMA,
        pltpu.SemaphoreType.DMA,
    ],
)
def _deg_kernel(src_h, dst_h, deg_o_h, deg_i_h,
                sidx, didx, ones_v, zeros_v, deg_o_s, deg_i_s,
                ls0, ls1, ls2, ls3, ld0, ld1, ld2, ld3,
                so0, so1, so2, so3, si0, si1, si2, si3):
    c = lax.axis_index("c")
    s = lax.axis_index("s")
    wid = s * NC + c
    for i in range(CH // 16):
        ones_v[pl.ds(i * 16, 16)] = jnp.ones((16,), jnp.float32)
        zeros_v[pl.ds(i * 16, 16)] = jnp.zeros((16,), jnp.float32)
    for j in range(RPT // CH):
        pltpu.sync_copy(zeros_v, deg_o_s.at[pl.ds(s * RPT + j * CH, CH)])
        pltpu.sync_copy(zeros_v, deg_i_s.at[pl.ds(s * RPT + j * CH, CH)])
    plsc.subcore_barrier()

    base0 = wid * EPW
    lsems = (ls0, ls1, ls2, ls3)
    ldems = (ld0, ld1, ld2, ld3)
    osems = (so0, so1, so2, so3)
    isems = (si0, si1, si2, si3)

    def load_idx(chunk, b):
        base = base0 + chunk * CH
        pltpu.async_copy(src_h.at[pl.ds(base, CH)], sidx.at[b], lsems[b])
        pltpu.async_copy(dst_h.at[pl.ds(base, CH)], didx.at[b], ldems[b])

    for b in range(2):
        load_idx(b, b)

    def body(i, carry):
        for b in range(4):
            ch = 4 * i + b
            pltpu.make_async_copy(src_h.at[pl.ds(base0, CH)], sidx.at[b], lsems[b]).wait()
            pltpu.make_async_copy(dst_h.at[pl.ds(base0, CH)], didx.at[b], ldems[b]).wait()
            pltpu.async_copy(ones_v, deg_o_s.at[sidx.at[b]], osems[b], add=True)
            pltpu.async_copy(ones_v, deg_i_s.at[didx.at[b]], isems[b], add=True)
            b2 = (b + 2) % 4

            @pl.when(ch + 2 < NCH)
            def _():
                @pl.when(ch >= 2)
                def _():
                    pltpu.make_async_copy(
                        ones_v, deg_o_s.at[sidx.at[b2]], osems[b2]).wait()
                    pltpu.make_async_copy(
                        ones_v, deg_i_s.at[didx.at[b2]], isems[b2]).wait()

                load_idx(ch + 2, b2)

        return carry

    lax.fori_loop(0, NCH // 4, body, 0)
    for b in range(4):
        pltpu.make_async_copy(ones_v, deg_o_s.at[sidx.at[b]], osems[b]).wait()
        pltpu.make_async_copy(ones_v, deg_i_s.at[didx.at[b]], isems[b]).wait()
    plsc.subcore_barrier()
    pltpu.sync_copy(deg_o_s.at[pl.ds(s * RPT, RPT)],
                    deg_o_h.at[pl.ds(c * N_PAD + s * RPT, RPT)])
    pltpu.sync_copy(deg_i_s.at[pl.ds(s * RPT, RPT)],
                    deg_i_h.at[pl.ds(c * N_PAD + s * RPT, RPT)])


@functools.partial(
    pl.kernel,
    mesh=_mesh,
    out_type=jax.ShapeDtypeStruct((NC, N_PAD, D), jnp.float32),
    scratch_types=[
        pltpu.VMEM((4, CHA), jnp.int32),
        pltpu.VMEM((4, CHA), jnp.int32),
        pltpu.VMEM((4, CHA, D), jnp.float32),
        pltpu.VMEM_SHARED((N_PAD, D), jnp.float32),
        pltpu.SemaphoreType.DMA,
        pltpu.SemaphoreType.DMA,
        pltpu.SemaphoreType.DMA,
        pltpu.SemaphoreType.DMA,
        pltpu.SemaphoreType.DMA,
        pltpu.SemaphoreType.DMA,
        pltpu.SemaphoreType.DMA,
        pltpu.SemaphoreType.DMA,
    ],
)
def _agg_kernel(h_h, src_h, dst_h, zeros_h, out_h, sidx, didx, rows, accum,
                g0, g1, g2, g3, s0, s1, s2, s3):
    c = lax.axis_index("c")
    s = lax.axis_index("s")
    wid = s * NC + c
    pltpu.sync_copy(zeros_h.at[pl.ds(s * RPT, RPT)], accum.at[pl.ds(s * RPT, RPT)])
    plsc.subcore_barrier()

    is0 = c == 0
    base0 = jnp.where(is0, s * EPW0, OFF1 + s * EPW1)
    nch = jnp.where(is0, EPW0 // CHA, EPW1 // CHA)
    gsems = (g0, g1, g2, g3)
    ssems = (s0, s1, s2, s3)

    def load_and_gather(chunk, b):
        base = base0 + chunk * CHA
        pltpu.sync_copy(src_h.at[pl.ds(base, CHA)], sidx.at[b])
        pltpu.sync_copy(dst_h.at[pl.ds(base, CHA)], didx.at[b])
        pltpu.async_copy(h_h.at[sidx.at[b]], rows.at[b], gsems[b])

    for b in range(2):
        load_and_gather(b, b)

    def body(i, carry):
        for b in range(4):
            ch = 4 * i + b
            pltpu.make_async_copy(h_h.at[sidx.at[b]], rows.at[b], gsems[b]).wait()
            pltpu.async_copy(rows.at[b], accum.at[didx.at[b]], ssems[b], add=True)
            b2 = (b + 2) % 4

            @pl.when(ch + 2 < nch)
            def _():
                @pl.when(ch >= 2)
                def _():
                    pltpu.make_async_copy(
                        rows.at[b2], accum.at[didx.at[b2]], ssems[b2]).wait()

                load_and_gather(ch + 2, b2)

        return carry

    lax.fori_loop(0, nch // 4, body, 0)
    for b in range(4):
        pltpu.make_async_copy(rows.at[b], accum.at[didx.at[b]], ssems[b]).wait()
    plsc.subcore_barrier()
    pltpu.sync_copy(accum.at[pl.ds(s * RPT, RPT)], out_h.at[c, pl.ds(s * RPT, RPT)])


def _tc1_body(x_ref, do0, do1, di0, di1, hs_ref, ns_ref, nd_ref):
    deg_o = do0[0] + do1[0]
    deg_i = di0[0] + di1[0]
    ns = lax.rsqrt(jnp.maximum(deg_o, 1.0))
    nd = lax.rsqrt(jnp.maximum(deg_i, 1.0))
    ns_ref[...] = ns
    nd_ref[...] = nd
    hs_ref[...] = x_ref[...] * ns


def _tc1(x, deg_o_p, deg_i_p):
    return pl.pallas_call(
        _tc1_body,
        grid=(GRID,),
        in_specs=[
            pl.BlockSpec((BLK, D), lambda i: (i, 0)),
            pl.BlockSpec((1, BLK, 1), lambda i: (0, i, 0)),
            pl.BlockSpec((1, BLK, 1), lambda i: (1, i, 0)),
            pl.BlockSpec((1, BLK, 1), lambda i: (0, i, 0)),
            pl.BlockSpec((1, BLK, 1), lambda i: (1, i, 0)),
        ],
        out_specs=[
            pl.BlockSpec((BLK, D), lambda i: (i, 0)),
            pl.BlockSpec((BLK, 1), lambda i: (i, 0)),
            pl.BlockSpec((BLK, 1), lambda i: (i, 0)),
        ],
        out_shape=[
            jax.ShapeDtypeStruct((N, D), jnp.float32),
            jax.ShapeDtypeStruct((N, 1), jnp.float32),
            jax.ShapeDtypeStruct((N, 1), jnp.float32),
        ],
    )(x, deg_o_p, deg_o_p, deg_i_p, deg_i_p)


def _tc2_body(p_ref0, p_ref1, nd_ref, ns_ref, w_ref, b_ref, out_ref):
    m = (p_ref0[0] + p_ref1[0]) * nd_ref[...]
    h = jnp.dot(m, w_ref[...], preferred_element_type=jnp.float32) + b_ref[...]
    out_ref[...] = jnp.maximum(h, 0.0) * ns_ref[...]


def _tc2(m_p, nd, ns, w, b):
    return pl.pallas_call(
        _tc2_body,
        grid=(GRID,),
        in_specs=[
            pl.BlockSpec((1, BLK, D), lambda i: (0, i, 0)),
            pl.BlockSpec((1, BLK, D), lambda i: (1, i, 0)),
            pl.BlockSpec((BLK, 1), lambda i: (i, 0)),
            pl.BlockSpec((BLK, 1), lambda i: (i, 0)),
            pl.BlockSpec((D, D), lambda i: (0, 0)),
            pl.BlockSpec((1, D), lambda i: (0, 0)),
        ],
        out_specs=pl.BlockSpec((BLK, D), lambda i: (i, 0)),
        out_shape=jax.ShapeDtypeStruct((N, D), jnp.float32),
    )(m_p, m_p, nd, ns, w, b)


def _tc3_body(p_ref0, p_ref1, nd_ref, w_ref, b_ref, out_ref, acc_vec, acc_nrm):
    i = pl.program_id(0)
    m = (p_ref0[0] + p_ref1[0]) * nd_ref[...]
    h = jnp.dot(m, w_ref[...], preferred_element_type=jnp.float32) + b_ref[...]
    blk_sum = jnp.sum(h, axis=0, keepdims=True)
    blk_nrm = jnp.sum(jnp.sqrt(jnp.sum(h * h, axis=1)))

    @pl.when(i == 0)
    def _():
        acc_vec[...] = jnp.zeros_like(acc_vec)
        acc_nrm[0] = 0.0

    acc_vec[...] += blk_sum
    acc_nrm[0] += blk_nrm

    @pl.when(i == pl.num_programs(0) - 1)
    def _():
        factor = jnp.sqrt(jnp.float32(D)) * jnp.float32(N) / acc_nrm[0]
        out_ref[...] = acc_vec[...] * factor


def _tc3(m_p, nd, w, b):
    return pl.pallas_call(
        _tc3_body,
        grid=(GRID,),
        in_specs=[
            pl.BlockSpec((1, BLK, D), lambda i: (0, i, 0)),
            pl.BlockSpec((1, BLK, D), lambda i: (1, i, 0)),
            pl.BlockSpec((BLK, 1), lambda i: (i, 0)),
            pl.BlockSpec((D, D), lambda i: (0, 0)),
            pl.BlockSpec((1, D), lambda i: (0, 0)),
        ],
        out_specs=pl.BlockSpec((1, D), lambda i: (0, 0)),
        out_shape=jax.ShapeDtypeStruct((1, D), jnp.float32),
        scratch_shapes=[
            pltpu.VMEM((1, D), jnp.float32),
            pltpu.SMEM((1,), jnp.float32),
        ],
    )(m_p, m_p, nd, w, b)


def kernel(x, edge_index, W1, b1, W2, b2):
    ei = edge_index.astype(jnp.int32)
    src = ei[0]
    dst = ei[1]
    pad = E_PAD - E
    src_g = jnp.concatenate([src, jnp.zeros((pad,), jnp.int32)])
    src_d = jnp.concatenate([src, jnp.full((pad,), N, jnp.int32)])
    dst_p = jnp.concatenate([dst, jnp.full((pad,), N, jnp.int32)])

    zeros_d = jnp.zeros((N_PAD, D), jnp.float32)

    deg_o_f, deg_i_f = _deg_kernel(src_d, dst_p)
    deg_o_p = deg_o_f.reshape(NC, N_PAD, 1)
    deg_i_p = deg_i_f.reshape(NC, N_PAD, 1)
    h1s, ns, nd = _tc1(x, deg_o_p, deg_i_p)
    m1_p = _agg_kernel(h1s, src_g, dst_p, zeros_d)
    h2s = _tc2(m1_p, nd, ns, W1, b1.reshape(1, D))
    m2_p = _agg_kernel(h2s, src_g, dst_p, zeros_d)
    out = _tc3(m2_p, nd, W2, b2.reshape(1, D))
    return out


# speedup vs baseline: 3.2438x; 1.0070x over previous
import functools

import jax
import jax.numpy as jnp
from jax import lax
from jax.experimental import pallas as pl
from jax.experimental.pallas import tpu as pltpu
from jax.experimental.pallas import tpu_sc as plsc

N = 10000
D = 128
E = 320000

NC = 2
NS = 16
NW = NC * NS
CH = 128
EPW = 10240
E_PAD = EPW * NW
NCH = EPW // CH
CHA = 64
NCHA = EPW // CHA
EPW0 = 16384
EPW1 = 4096
OFF1 = NS * EPW0
N_PAD = 10240
RPT = N_PAD // NS

BLK = 1000
GRID = N // BLK

_mesh = plsc.VectorSubcoreMesh(core_axis_name="c", subcore_axis_name="s")


@functools.partial(
    pl.kernel,
    mesh=_mesh,
    out_type=(
        jax.ShapeDtypeStruct((NC * N_PAD,), jnp.float32),
        jax.ShapeDtypeStruct((NC * N_PAD,), jnp.float32),
    ),
    scratch_types=[
        pltpu.VMEM((4, CH), jnp.int32),
        pltpu.VMEM((4, CH), jnp.int32),
        pltpu.VMEM((CH,), jnp.float32),
        pltpu.VMEM((CH,), jnp.float32),
        pltpu.VMEM_SHARED((N_PAD,), jnp.float32),
        pltpu.VMEM_SHARED((N_PAD,), jnp.float32),
        pltpu.SemaphoreType.DMA,
        pltpu.SemaphoreType.DMA,
        pltpu.SemaphoreType.DMA,
        pltpu.SemaphoreType.DMA,
        pltpu.SemaphoreType.DMA,
        pltpu.SemaphoreType.DMA,
        pltpu.SemaphoreType.DMA,
        pltpu.SemaphoreType.DMA,
        pltpu.SemaphoreType.DMA,
        pltpu.SemaphoreType.DMA,
        pltpu.SemaphoreType.DMA,
        pltpu.SemaphoreType.DMA,
        pltpu.SemaphoreType.DMA,
        pltpu.SemaphoreType.DMA,
        pltpu.SemaphoreType.DMA,
        pltpu.SemaphoreType.DMA,
    ],
)
def _deg_kernel(src_h, dst_h, deg_o_h, deg_i_h,
                sidx, didx, ones_v, zeros_v, deg_o_s, deg_i_s,
                ls0, ls1, ls2, ls3, ld0, ld1, ld2, ld3,
                so0, so1, so2, so3, si0, si1, si2, si3):
    c = lax.axis_index("c")
    s = lax.axis_index("s")
    wid = s * NC + c
    for i in range(CH // 16):
        ones_v[pl.ds(i * 16, 16)] = jnp.ones((16,), jnp.float32)
        zeros_v[pl.ds(i * 16, 16)] = jnp.zeros((16,), jnp.float32)
    for j in range(RPT // CH):
        pltpu.sync_copy(zeros_v, deg_o_s.at[pl.ds(s * RPT + j * CH, CH)])
        pltpu.sync_copy(zeros_v, deg_i_s.at[pl.ds(s * RPT + j * CH, CH)])
    plsc.subcore_barrier()

    base0 = wid * EPW
    lsems = (ls0, ls1, ls2, ls3)
    ldems = (ld0, ld1, ld2, ld3)
    osems = (so0, so1, so2, so3)
    isems = (si0, si1, si2, si3)

    def load_idx(chunk, b):
        base = base0 + chunk * CH
        pltpu.async_copy(src_h.at[pl.ds(base, CH)], sidx.at[b], lsems[b])
        pltpu.async_copy(dst_h.at[pl.ds(base, CH)], didx.at[b], ldems[b])

    for b in range(2):
        load_idx(b, b)

    def body(i, carry):
        for b in range(4):
            ch = 4 * i + b
            pltpu.make_async_copy(src_h.at[pl.ds(base0, CH)], sidx.at[b], lsems[b]).wait()
            pltpu.make_async_copy(dst_h.at[pl.ds(base0, CH)], didx.at[b], ldems[b]).wait()
            pltpu.async_copy(ones_v, deg_o_s.at[sidx.at[b]], osems[b], add=True)
            pltpu.async_copy(ones_v, deg_i_s.at[didx.at[b]], isems[b], add=True)
            b2 = (b + 2) % 4

            @pl.when(ch + 2 < NCH)
            def _():
                @pl.when(ch >= 2)
                def _():
                    pltpu.make_async_copy(
                        ones_v, deg_o_s.at[sidx.at[b2]], osems[b2]).wait()
                    pltpu.make_async_copy(
                        ones_v, deg_i_s.at[didx.at[b2]], isems[b2]).wait()

                load_idx(ch + 2, b2)

        return carry

    lax.fori_loop(0, NCH // 4, body, 0)
    for b in range(4):
        pltpu.make_async_copy(ones_v, deg_o_s.at[sidx.at[b]], osems[b]).wait()
        pltpu.make_async_copy(ones_v, deg_i_s.at[didx.at[b]], isems[b]).wait()
    plsc.subcore_barrier()
    pltpu.sync_copy(deg_o_s.at[pl.ds(s * RPT, RPT)],
                    deg_o_h.at[pl.ds(c * N_PAD + s * RPT, RPT)])
    pltpu.sync_copy(deg_i_s.at[pl.ds(s * RPT, RPT)],
                    deg_i_h.at[pl.ds(c * N_PAD + s * RPT, RPT)])


@functools.partial(
    pl.kernel,
    mesh=_mesh,
    out_type=jax.ShapeDtypeStruct((NC, N_PAD, D), jnp.float32),
    scratch_types=[
        pltpu.VMEM((4, CHA), jnp.int32),
        pltpu.VMEM((4, CHA), jnp.int32),
        pltpu.VMEM((4, CHA, D), jnp.float32),
        pltpu.VMEM_SHARED((N_PAD, D), jnp.float32),
        pltpu.SemaphoreType.DMA,
        pltpu.SemaphoreType.DMA,
        pltpu.SemaphoreType.DMA,
        pltpu.SemaphoreType.DMA,
        pltpu.SemaphoreType.DMA,
        pltpu.SemaphoreType.DMA,
        pltpu.SemaphoreType.DMA,
        pltpu.SemaphoreType.DMA,
    ],
)
def _agg_kernel(h_h, src_h, dst_h, zeros_h, out_h, sidx, didx, rows, accum,
                g0, g1, g2, g3, s0, s1, s2, s3):
    c = lax.axis_index("c")
    s = lax.axis_index("s")
    wid = s * NC + c
    pltpu.sync_copy(zeros_h.at[pl.ds(s * RPT, RPT)], accum.at[pl.ds(s * RPT, RPT)])
    plsc.subcore_barrier()

    is0 = c == 0
    base0 = jnp.where(is0, s * EPW0, OFF1 + s * EPW1)
    nch = jnp.where(is0, EPW0 // CHA, EPW1 // CHA)
    gsems = (g0, g1, g2, g3)
    ssems = (s0, s1, s2, s3)

    def load_and_gather(chunk, b):
        base = base0 + chunk * CHA
        pltpu.sync_copy(src_h.at[pl.ds(base, CHA)], sidx.at[b])
        pltpu.sync_copy(dst_h.at[pl.ds(base, CHA)], didx.at[b])
        pltpu.async_copy(h_h.at[sidx.at[b]], rows.at[b], gsems[b])

    for b in range(2):
        load_and_gather(b, b)

    def body(i, carry):
        for b in range(4):
            ch = 4 * i + b
            pltpu.make_async_copy(h_h.at[sidx.at[b]], rows.at[b], gsems[b]).wait()
            pltpu.async_copy(rows.at[b], accum.at[didx.at[b]], ssems[b], add=True)
            b2 = (b + 2) % 4

            @pl.when(ch + 2 < nch)
            def _():
                @pl.when(ch >= 2)
                def _():
                    pltpu.make_async_copy(
                        rows.at[b2], accum.at[didx.at[b2]], ssems[b2]).wait()

                load_and_gather(ch + 2, b2)

        return carry

    lax.fori_loop(0, nch // 4, body, 0)
    for b in range(4):
        pltpu.make_async_copy(rows.at[b], accum.at[didx.at[b]], ssems[b]).wait()
    plsc.subcore_barrier()
    pltpu.sync_copy(accum.at[pl.ds(s * RPT, RPT)], out_h.at[c, pl.ds(s * RPT, RPT)])


def _tc1_body(x_ref, do0, do1, di0, di1, hs_ref, ns_ref, nd_ref):
    deg_o = do0[0] + do1[0]
    deg_i = di0[0] + di1[0]
    ns = lax.rsqrt(jnp.maximum(deg_o, 1.0))
    nd = lax.rsqrt(jnp.maximum(deg_i, 1.0))
    ns_ref[...] = ns
    nd_ref[...] = nd
    hs_ref[...] = x_ref[...] * ns


def _tc1(x, deg_o_p, deg_i_p):
    return pl.pallas_call(
        _tc1_body,
        grid=(GRID,),
        in_specs=[
            pl.BlockSpec((BLK, D), lambda i: (i, 0)),
            pl.BlockSpec((1, BLK, 1), lambda i: (0, i, 0)),
            pl.BlockSpec((1, BLK, 1), lambda i: (1, i, 0)),
            pl.BlockSpec((1, BLK, 1), lambda i: (0, i, 0)),
            pl.BlockSpec((1, BLK, 1), lambda i: (1, i, 0)),
        ],
        out_specs=[
            pl.BlockSpec((BLK, D), lambda i: (i, 0)),
            pl.BlockSpec((BLK, 1), lambda i: (i, 0)),
            pl.BlockSpec((BLK, 1), lambda i: (i, 0)),
        ],
        out_shape=[
            jax.ShapeDtypeStruct((N, D), jnp.float32),
            jax.ShapeDtypeStruct((N, 1), jnp.float32),
            jax.ShapeDtypeStruct((N, 1), jnp.float32),
        ],
    )(x, deg_o_p, deg_o_p, deg_i_p, deg_i_p)


def _tc2_body(p_ref0, p_ref1, nd_ref, ns_ref, w_ref, b_ref, out_ref):
    m = (p_ref0[0] + p_ref1[0]) * nd_ref[...]
    h = jnp.dot(m, w_ref[...], preferred_element_type=jnp.float32) + b_ref[...]
    out_ref[...] = jnp.maximum(h, 0.0) * ns_ref[...]


def _tc2(m_p, nd, ns, w, b):
    return pl.pallas_call(
        _tc2_body,
        grid=(GRID,),
        in_specs=[
            pl.BlockSpec((1, BLK, D), lambda i: (0, i, 0)),
            pl.BlockSpec((1, BLK, D), lambda i: (1, i, 0)),
            pl.BlockSpec((BLK, 1), lambda i: (i, 0)),
            pl.BlockSpec((BLK, 1), lambda i: (i, 0)),
            pl.BlockSpec((D, D), lambda i: (0, 0)),
            pl.BlockSpec((1, D), lambda i: (0, 0)),
        ],
        out_specs=pl.BlockSpec((BLK, D), lambda i: (i, 0)),
        out_shape=jax.ShapeDtypeStruct((N, D), jnp.float32),
    )(m_p, m_p, nd, ns, w, b)


def _tc3_body(p_ref0, p_ref1, nd_ref, w_ref, b_ref, out_ref, acc_vec, acc_nrm):
    i = pl.program_id(0)
    m = (p_ref0[0] + p_ref1[0]) * nd_ref[...]
    h = jnp.dot(m, w_ref[...], preferred_element_type=jnp.float32) + b_ref[...]
    blk_sum = jnp.sum(h, axis=0, keepdims=True)
    blk_nrm = jnp.sum(jnp.sqrt(jnp.sum(h * h, axis=1)))

    @pl.when(i == 0)
    def _():
        acc_vec[...] = jnp.zeros_like(acc_vec)
        acc_nrm[0] = 0.0

    acc_vec[...] += blk_sum
    acc_nrm[0] += blk_nrm

    @pl.when(i == pl.num_programs(0) - 1)
    def _():
        factor = jnp.sqrt(jnp.float32(D)) * jnp.float32(N) / acc_nrm[0]
        out_ref[...] = acc_vec[...] * factor


def _tc3(m_p, nd, w, b):
    return pl.pallas_call(
        _tc3_body,
        grid=(GRID,),
        in_specs=[
            pl.BlockSpec((1, BLK, D), lambda i: (0, i, 0)),
            pl.BlockSpec((1, BLK, D), lambda i: (1, i, 0)),
            pl.BlockSpec((BLK, 1), lambda i: (i, 0)),
            pl.BlockSpec((D, D), lambda i: (0, 0)),
            pl.BlockSpec((1, D), lambda i: (0, 0)),
        ],
        out_specs=pl.BlockSpec((1, D), lambda i: (0, 0)),
        out_shape=jax.ShapeDtypeStruct((1, D), jnp.float32),
        scratch_shapes=[
            pltpu.VMEM((1, D), jnp.float32),
            pltpu.SMEM((1,), jnp.float32),
        ],
    )(m_p, m_p, nd, w, b)


def kernel(x, edge_index, W1, b1, W2, b2):
    ei = edge_index.astype(jnp.int32)
    src = ei[0]
    dst = ei[1]
    pad = E_PAD - E
    src_g = jnp.concatenate([src, jnp.zeros((pad,), jnp.int32)])
    src_d = jnp.concatenate([src, jnp.full((pad,), N, jnp.int32)])
    dst_p = jnp.concatenate([dst, jnp.full((pad,), N, jnp.int32)])

    zeros_d = jnp.zeros((N_PAD, D), jnp.float32)

    deg_o_f, deg_i_f = _deg_kernel(src_d, dst_p)
    deg_o_p = deg_o_f.reshape(NC, N_PAD, 1)
    deg_i_p = deg_i_f.reshape(NC, N_PAD, 1)
    h1s, ns, nd = _tc1(x, deg_o_p, deg_i_p)
    m1_p = _agg_kernel(h1s, src_g, dst_p, zeros_d)
    h2s = _tc2(m1_p, nd, ns, W1, b1.reshape(1, D))
    m2_p = _agg_kernel(h2s, src_g, dst_p, zeros_d)
    out = _tc3(m2_p, nd, W2, b2.reshape(1, D))
    return out
